# Initial kernel scaffold; baseline (speedup 1.0000x reference)
import jax
import jax.numpy as jnp
from jax.experimental import pallas as pl


def kernel(features, img_feat, gene_edge, img_edge, gene_attr, img_attr, params):
    raise NotImplementedError("write your pallas kernel here")


# speedup vs baseline: 10.4306x; 10.4306x over previous
import functools

import jax
import jax.numpy as jnp
from jax.experimental import pallas as pl
from jax.experimental.pallas import tpu as pltpu
from jax.experimental.pallas import tpu_sc as plsc

_C = 128
_NT = 32


def _mesh():
    return plsc.VectorSubcoreMesh(core_axis_name="c", subcore_axis_name="s")


def _sc_degrees(cols_g, attr_g, cols_i, attr_i, zN):
    E = cols_g.shape[0]
    Np = zN.shape[0]
    NCH = E // _C
    base_ch = NCH // _NT
    rem = NCH - base_ch * _NT

    def body(cg_h, ag_h, ci_h, ai_h, z_h, oga, og1, oia, oi1,
             cidx, vbuf, ones, acc_ga, acc_g1, acc_ia, acc_i1):
        cid = jax.lax.axis_index("c")
        sid = jax.lax.axis_index("s")
        t = sid * 2 + cid
        for i in range(_C // 16):
            ones[pl.ds(i * 16, 16)] = jnp.ones((16,), jnp.float32)
        @pl.when(sid == 0)
        def _():
            pltpu.sync_copy(z_h, acc_ga)
            pltpu.sync_copy(z_h, acc_g1)
            pltpu.sync_copy(z_h, acc_ia)
            pltpu.sync_copy(z_h, acc_i1)
        plsc.subcore_barrier()
        nch = base_ch + jnp.where(t < rem, 1, 0)

        def run_set(cols_h, attr_h, acc_a, acc_1):
            def chunk(i, carry):
                base = (t + i * _NT) * _C
                pltpu.sync_copy(cols_h.at[pl.ds(base, _C)], cidx)
                pltpu.sync_copy(attr_h.at[pl.ds(base, _C)], vbuf)
                pltpu.sync_copy(vbuf, acc_a.at[cidx], add=True)
                pltpu.sync_copy(ones, acc_1.at[cidx], add=True)
                return carry
            jax.lax.fori_loop(0, nch, chunk, 0)

        run_set(cg_h, ag_h, acc_ga, acc_g1)
        run_set(ci_h, ai_h, acc_ia, acc_i1)
        plsc.subcore_barrier()
        @pl.when(sid == 0)
        def _():
            pltpu.sync_copy(acc_ga, oga.at[cid])
            pltpu.sync_copy(acc_g1, og1.at[cid])
            pltpu.sync_copy(acc_ia, oia.at[cid])
            pltpu.sync_copy(acc_i1, oi1.at[cid])

    out = jax.ShapeDtypeStruct((2, Np), jnp.float32)
    scratch = [
        pltpu.VMEM((_C,), jnp.int32),
        pltpu.VMEM((_C,), jnp.float32),
        pltpu.VMEM((_C,), jnp.float32),
        pltpu.VMEM_SHARED((Np,), jnp.float32),
        pltpu.VMEM_SHARED((Np,), jnp.float32),
        pltpu.VMEM_SHARED((Np,), jnp.float32),
        pltpu.VMEM_SHARED((Np,), jnp.float32),
    ]
    return pl.kernel(body, out_type=(out,) * 4, mesh=_mesh(),
                     scratch_types=scratch)(cols_g, attr_g, cols_i, attr_i, zN)


def _sc_agg(rows, cols, y, zeros, ew=None):
    E = rows.shape[0]
    Np, D = y.shape
    NCH = E // _C
    base_ch = NCH // _NT
    rem = NCH - base_ch * _NT
    rpt = (Np // 16) & ~7
    tail = Np - 16 * rpt
    with_ew = ew is not None

    def body(*refs):
        if with_ew:
            (rows_h, cols_h, ew_h, y_h, z_h, out_h,
             ridx, cidx, buf, ewb, acc, sem) = refs
        else:
            (rows_h, cols_h, y_h, z_h, out_h,
             ridx, cidx, buf, acc, sem) = refs
        cid = jax.lax.axis_index("c")
        sid = jax.lax.axis_index("s")
        t = sid * 2 + cid
        pltpu.sync_copy(z_h.at[pl.ds(sid * rpt, rpt)],
                        acc.at[pl.ds(sid * rpt, rpt)])
        if tail:
            @pl.when(sid == 15)
            def _():
                pltpu.sync_copy(z_h.at[pl.ds(16 * rpt, tail)],
                                acc.at[pl.ds(16 * rpt, tail)])
        plsc.subcore_barrier()
        nch = base_ch + jnp.where(t < rem, 1, 0)

        def chunk(i, carry):
            base = (t + i * _NT) * _C
            pltpu.sync_copy(rows_h.at[pl.ds(base, _C)], ridx)
            pltpu.sync_copy(cols_h.at[pl.ds(base, _C)], cidx)
            pltpu.async_copy(y_h.at[ridx], buf, sem).wait()
            if with_ew:
                pltpu.sync_copy(ew_h.at[pl.ds(base, _C)], ewb)

                def escale(j, c2):
                    s = plsc.load_gather(ewb, [jnp.full((16,), j, jnp.int32)])
                    for d in range(D // 16):
                        sl = pl.ds(d * 16, 16)
                        buf[j, sl] = buf[j, sl] * s
                    return c2
                jax.lax.fori_loop(0, _C, escale, 0)
            pltpu.sync_copy(buf, acc.at[cidx], add=True)
            return carry
        jax.lax.fori_loop(0, nch, chunk, 0)
        plsc.subcore_barrier()
        pltpu.sync_copy(acc.at[pl.ds(sid * rpt, rpt)],
                        out_h.at[cid, pl.ds(sid * rpt, rpt)])
        if tail:
            @pl.when(sid == 15)
            def _():
                pltpu.sync_copy(acc.at[pl.ds(16 * rpt, tail)],
                                out_h.at[cid, pl.ds(16 * rpt, tail)])

    scratch = [pltpu.VMEM((_C,), jnp.int32), pltpu.VMEM((_C,), jnp.int32),
               pltpu.VMEM((_C, D), jnp.float32)]
    if with_ew:
        scratch.append(pltpu.VMEM((_C,), jnp.float32))
    scratch += [pltpu.VMEM_SHARED((Np, D), jnp.float32),
                pltpu.SemaphoreType.DMA]
    args = (rows, cols) + ((ew,) if with_ew else ()) + (y, zeros)
    cp = pltpu.CompilerParams(needs_layout_passes=not with_ew,
                              use_tc_tiling_on_sc=False)
    return pl.kernel(body, out_type=jax.ShapeDtypeStruct((2, Np, D), jnp.float32),
                     mesh=_mesh(), scratch_types=scratch,
                     compiler_params=cp)(*args)


_BR = 1000


def _tc_prep(d0, d1):
    def body(a_ref, b_ref, dinv_ref, dinv2_ref):
        deg = a_ref[...] + b_ref[...] + 1.0
        dinv2_ref[...] = 1.0 / deg
        dinv_ref[...] = jax.lax.rsqrt(deg)
    sds = jax.ShapeDtypeStruct(d0.shape, jnp.float32)
    return pl.pallas_call(body, out_shape=(sds, sds))(d0, d1)


def _tc_mm(x, W, dn):
    Np, K = x.shape
    Dn = W.shape[1]
    G = Np // _BR

    def body(x_ref, w_ref, dn_ref, xw_ref, y_ref):
        xw = jnp.dot(x_ref[...], w_ref[...], preferred_element_type=jnp.float32)
        xw_ref[...] = xw
        y_ref[...] = xw * dn_ref[...]

    sds = jax.ShapeDtypeStruct((Np, Dn), jnp.float32)
    return pl.pallas_call(
        body, grid=(G,),
        in_specs=[pl.BlockSpec((_BR, K), lambda i: (i, 0)),
                  pl.BlockSpec((K, Dn), lambda i: (0, 0)),
                  pl.BlockSpec((_BR, 1), lambda i: (i, 0))],
        out_specs=[pl.BlockSpec((_BR, Dn), lambda i: (i, 0))] * 2,
        out_shape=(sds, sds))(x, W, dn)


def _tc_mm2(x1, W1, x2, W2, dn):
    Np, K1 = x1.shape
    Dn = W1.shape[1]
    K2 = x2.shape[1]
    G = Np // _BR

    def body(x1_ref, w1_ref, x2_ref, w2_ref, dn_ref, xw_ref, y_ref):
        xw = (jnp.dot(x1_ref[...], w1_ref[...], preferred_element_type=jnp.float32)
              + jnp.dot(x2_ref[...], w2_ref[...], preferred_element_type=jnp.float32))
        xw_ref[...] = xw
        y_ref[...] = xw * dn_ref[...]

    sds = jax.ShapeDtypeStruct((Np, Dn), jnp.float32)
    return pl.pallas_call(
        body, grid=(G,),
        in_specs=[pl.BlockSpec((_BR, K1), lambda i: (i, 0)),
                  pl.BlockSpec((K1, Dn), lambda i: (0, 0)),
                  pl.BlockSpec((_BR, K2), lambda i: (i, 0)),
                  pl.BlockSpec((K2, Dn), lambda i: (0, 0)),
                  pl.BlockSpec((_BR, 1), lambda i: (i, 0))],
        out_specs=[pl.BlockSpec((_BR, Dn), lambda i: (i, 0))] * 2,
        out_shape=(sds, sds))(x1, W1, x2, W2, dn)


def _elu(h):
    return jnp.where(h > 0, h, jnp.exp(jnp.minimum(h, 0.0)) - 1.0)


def _tc_comb_mm(p, xw, di, di2, b, use_elu, Wn, dn, emit_h=False):
    Np, D = xw.shape
    Dn = Wn.shape[1]
    G = Np // _BR

    def body(p_ref, xw_ref, di_ref, di2_ref, b_ref, w_ref, dn_ref, *out_refs):
        h = (di_ref[...] * (p_ref[0] + p_ref[1])
             + di2_ref[...] * xw_ref[...] + b_ref[...])
        if emit_h:
            out_refs[0][...] = h
        g = _elu(h) if use_elu else h
        xwn = jnp.dot(g, w_ref[...], preferred_element_type=jnp.float32)
        out_refs[-2][...] = xwn
        out_refs[-1][...] = xwn * dn_ref[...]

    sds_h = jax.ShapeDtypeStruct((Np, D), jnp.float32)
    sds_n = jax.ShapeDtypeStruct((Np, Dn), jnp.float32)
    out_shape = ((sds_h,) if emit_h else ()) + (sds_n, sds_n)
    out_specs = (([pl.BlockSpec((_BR, D), lambda i: (i, 0))] if emit_h else [])
                 + [pl.BlockSpec((_BR, Dn), lambda i: (i, 0))] * 2)
    return pl.pallas_call(
        body, grid=(G,),
        in_specs=[pl.BlockSpec((2, _BR, D), lambda i: (0, i, 0)),
                  pl.BlockSpec((_BR, D), lambda i: (i, 0)),
                  pl.BlockSpec((_BR, 1), lambda i: (i, 0)),
                  pl.BlockSpec((_BR, 1), lambda i: (i, 0)),
                  pl.BlockSpec((1, D), lambda i: (0, 0)),
                  pl.BlockSpec((D, Dn), lambda i: (0, 0)),
                  pl.BlockSpec((_BR, 1), lambda i: (i, 0))],
        out_specs=out_specs,
        out_shape=out_shape)(p, xw, di, di2, b, Wn, dn)


def _tc_comb_final(p, xw, di, di2, b):
    Np, D = xw.shape
    G = Np // _BR

    def body(p_ref, xw_ref, di_ref, di2_ref, b_ref, h_ref):
        h_ref[...] = (di_ref[...] * (p_ref[0] + p_ref[1])
                      + di2_ref[...] * xw_ref[...] + b_ref[...])

    return pl.pallas_call(
        body, grid=(G,),
        in_specs=[pl.BlockSpec((2, _BR, D), lambda i: (0, i, 0)),
                  pl.BlockSpec((_BR, D), lambda i: (i, 0)),
                  pl.BlockSpec((_BR, 1), lambda i: (i, 0)),
                  pl.BlockSpec((_BR, 1), lambda i: (i, 0)),
                  pl.BlockSpec((1, D), lambda i: (0, 0))],
        out_specs=pl.BlockSpec((_BR, D), lambda i: (i, 0)),
        out_shape=jax.ShapeDtypeStruct((Np, D), jnp.float32))(p, xw, di, di2, b)


def kernel(features, img_feat, gene_edge, img_edge, gene_attr, img_attr, params):
    Np = features.shape[0]
    P = params
    rows_g, cols_g = gene_edge[0], gene_edge[1]
    rows_i, cols_i = img_edge[0], img_edge[1]
    zN = jnp.zeros((Np,), jnp.float32)
    z128 = jnp.zeros((Np, 128), jnp.float32)
    z64 = jnp.zeros((Np, 64), jnp.float32)

    dga_p, dg1_p, dia_p, di1_p = _sc_degrees(cols_g, gene_attr, cols_i, img_attr, zN)
    d0 = jnp.stack([dga_p[0], dg1_p[0], dia_p[0], di1_p[0]])
    d1 = jnp.stack([dga_p[1], dg1_p[1], dia_p[1], di1_p[1]])
    dinv, dinv2 = _tc_prep(d0, d1)
    di_ga, di2_ga = dinv[0].reshape(Np, 1), dinv2[0].reshape(Np, 1)
    di_g1, di2_g1 = dinv[1].reshape(Np, 1), dinv2[1].reshape(Np, 1)
    di_ia, di2_ia = dinv[2].reshape(Np, 1), dinv2[2].reshape(Np, 1)
    di_i1, di2_i1 = dinv[3].reshape(Np, 1), dinv2[3].reshape(Np, 1)

    def wb(name):
        return P[name]["W"], P[name]["b"].reshape(1, -1)

    W1, b1 = wb("conv1"); W2, b2 = wb("conv2"); W3, b3 = wb("conv3"); W4, b4 = wb("conv4")
    xw1, y1 = _tc_mm(features, W1, di_ga)
    p1 = _sc_agg(rows_g, cols_g, y1, z128, ew=gene_attr)
    xw2, y2 = _tc_comb_mm(p1, xw1, di_ga, di2_ga, b1, True, W2, di_ga)
    p2 = _sc_agg(rows_g, cols_g, y2, z64, ew=gene_attr)
    h2, xw3, y3 = _tc_comb_mm(p2, xw2, di_ga, di2_ga, b2, False, W3, di_g1, emit_h=True)
    p3 = _sc_agg(rows_g, cols_g, y3, z128)
    xw4, y4 = _tc_comb_mm(p3, xw3, di_g1, di2_g1, b3, True, W4, di_g1)
    p4 = _sc_agg(rows_g, cols_g, y4, z128)
    h4 = _tc_comb_final(p4, xw4, di_g1, di2_g1, b4)

    Wi1, bi1 = wb("imgconv1"); Wi2, bi2 = wb("imgconv2")
    Wi3, bi3 = wb("imgconv3"); Wi4, bi4 = wb("imgconv4")
    xwi1, yi1 = _tc_mm(img_feat, Wi1, di_ia)
    q1 = _sc_agg(rows_i, cols_i, yi1, z128, ew=img_attr)
    xwi2, yi2 = _tc_comb_mm(q1, xwi1, di_ia, di2_ia, bi1, True, Wi2, di_ia)
    q2 = _sc_agg(rows_i, cols_i, yi2, z64, ew=img_attr)
    i2, xwi3, yi3 = _tc_comb_mm(q2, xwi2, di_ia, di2_ia, bi2, False, Wi3, di_i1, emit_h=True)
    q3 = _sc_agg(rows_i, cols_i, yi3, z128)
    xwi4, yi4 = _tc_comb_mm(q3, xwi3, di_i1, di2_i1, bi3, True, Wi4, di_i1)
    q4 = _sc_agg(rows_i, cols_i, yi4, z128)
    i4 = _tc_comb_final(q4, xwi4, di_i1, di2_i1, bi4)

    Wn, bn = wb("neck"); Wn2, bn2 = wb("neck2"); Wc3, bc3 = wb("c3"); Wc4, bc4 = wb("c4")
    xwn, yn = _tc_mm2(h2, Wn[:64], i2, Wn[64:], di_g1)
    pn = _sc_agg(rows_g, cols_g, yn, z64)
    xwn2, yn2 = _tc_comb_mm(pn, xwn, di_g1, di2_g1, bn, True, Wn2, di_g1)
    pn2 = _sc_agg(rows_g, cols_g, yn2, z64)
    c2, xwc3, yc3 = _tc_comb_mm(pn2, xwn2, di_g1, di2_g1, bn2, False, Wc3, di_g1, emit_h=True)
    pc3 = _sc_agg(rows_g, cols_g, yc3, z128)
    xwc4, yc4 = _tc_comb_mm(pc3, xwc3, di_g1, di2_g1, bc3, True, Wc4, di_g1)
    pc4 = _sc_agg(rows_g, cols_g, yc4, z128)
    c4 = _tc_comb_final(pc4, xwc4, di_g1, di2_g1, bc4)

    return (h2, i2, c2, h4, i4, c4)
